# baseline (speedup 1.0000x reference)
import functools

import jax
import jax.numpy as jnp
from jax import lax
from jax.experimental import pallas as pl
from jax.experimental.pallas import tpu as pltpu
from jax.experimental.pallas import tpu_sc as plsc

B = 16384
D = 128
NC = 2
NS = 16
L = 16
NW = NC * NS
BPW = B // NW
CH = 128
NCHUNK = BPW // CH

_mesh = plsc.VectorSubcoreMesh(core_axis_name="c", subcore_axis_name="s")


@functools.partial(
    pl.kernel,
    mesh=_mesh,
    compiler_params=pltpu.CompilerParams(needs_layout_passes=False),
    out_type=jax.ShapeDtypeStruct((B,), jnp.float32),
    scratch_types=[
        pltpu.VMEM((NCHUNK, CH), jnp.int32),
        pltpu.VMEM((NCHUNK, CH), jnp.int32),
        pltpu.VMEM((CH, D), jnp.float32),
        pltpu.VMEM((CH, D), jnp.float32),
        pltpu.VMEM((BPW,), jnp.float32),
        pltpu.SemaphoreType.DMA,
    ],
)
def _mf_sc(user_hbm, item_hbm, uemb_hbm, iemb_hbm, out_hbm,
           uidx_v, iidx_v, urows_v, irows_v, outv, sem):
    wid = lax.axis_index("s") * NC + lax.axis_index("c")
    base = wid * BPW

    for c in range(NCHUNK):
        pltpu.sync_copy(user_hbm.at[pl.ds(base + c * CH, CH)], uidx_v.at[c])
        pltpu.sync_copy(item_hbm.at[pl.ds(base + c * CH, CH)], iidx_v.at[c])

    lane = lax.iota(jnp.int32, L)

    for c in range(NCHUNK):
        pltpu.async_copy(uemb_hbm.at[uidx_v.at[c]], urows_v, sem).wait()
        pltpu.async_copy(iemb_hbm.at[iidx_v.at[c]], irows_v, sem).wait()

        def group_body(g, carry, c=c):
            vec = jnp.zeros((L,), jnp.float32)
            for k in range(L):
                r = g * L + k
                part = jnp.zeros((L,), jnp.float32)
                for j in range(D // L):
                    uu = urows_v[r, pl.ds(j * L, L)]
                    vv = irows_v[r, pl.ds(j * L, L)]
                    part = part + uu * vv
                vec = jnp.where(lane == k, jnp.sum(part), vec)
            outv[pl.ds(c * CH + g * L, L)] = vec
            return carry

        lax.fori_loop(0, CH // L, group_body, 0)

    pltpu.sync_copy(outv, out_hbm.at[pl.ds(base, BPW)])


def kernel(user, item, users_emb, items_emb):
    return _mf_sc(user, item, users_emb, items_emb)
